# Initial kernel scaffold; baseline (speedup 1.0000x reference)
import functools

import jax
import jax.numpy as jnp
from jax import lax
from jax.experimental import pallas as pl
from jax.experimental.pallas import tpu as pltpu
from jax.experimental.pallas import tpu_sc as plsc

N_NODES = 10000
N_EDGES = 160000
D_FEAT = 256
D_HALF = 128
N_CORES = 2
N_TILES = 16

EDGES_PER_TILE = N_EDGES // N_TILES
BLOCK_E = 80
N_BLOCKS = EDGES_PER_TILE // BLOCK_E
ROWS_PER_TILE = N_NODES // N_TILES


def _weight_body(x_ref, cj_ref, w_ref):
    w_ref[0] = x_ref[:, :D_HALF] * cj_ref[...]
    w_ref[1] = x_ref[:, D_HALF:] * cj_ref[...]


_R = 1000

_tc_weight = pl.pallas_call(
    _weight_body,
    grid=(N_NODES // _R,),
    in_specs=[
        pl.BlockSpec((_R, D_FEAT), lambda i: (i, 0)),
        pl.BlockSpec((_R, 1), lambda i: (i, 0)),
    ],
    out_specs=pl.BlockSpec((N_CORES, _R, D_HALF), lambda i: (0, i, 0)),
    out_shape=jax.ShapeDtypeStruct((N_CORES, N_NODES, D_HALF), jnp.float32),
)


def _sc_body(w_hbm, src_hbm, dst_hbm, zero_hbm, out_hbm,
             src_v, dst_v, rows_v, acc, sem):
    c = lax.axis_index("c")
    s = lax.axis_index("s")
    row_off = c * N_NODES
    edge_base = s * EDGES_PER_TILE

    r0 = s * ROWS_PER_TILE
    pltpu.sync_copy(zero_hbm.at[pl.ds(r0, ROWS_PER_TILE)],
                    acc.at[pl.ds(r0, ROWS_PER_TILE)])
    plsc.subcore_barrier()

    def block(i, carry):
        off = edge_base + i * BLOCK_E
        pltpu.sync_copy(src_hbm.at[pl.ds(off, BLOCK_E)], src_v)
        pltpu.sync_copy(dst_hbm.at[pl.ds(off, BLOCK_E)], dst_v)
        for j in range(BLOCK_E // 16):
            sl = pl.ds(j * 16, 16)
            src_v[sl] = src_v[sl] + row_off
        pltpu.async_copy(w_hbm.at[src_v], rows_v, sem).wait()
        pltpu.sync_copy(rows_v, acc.at[dst_v], add=True)
        return carry

    lax.fori_loop(0, N_BLOCKS, block, 0)

    plsc.subcore_barrier()
    pltpu.sync_copy(acc.at[pl.ds(r0, ROWS_PER_TILE)],
                    out_hbm.at[pl.ds(row_off + r0, ROWS_PER_TILE)])


_sc_gather_scatter = functools.partial(
    pl.kernel,
    out_type=jax.ShapeDtypeStruct((N_CORES * N_NODES, D_HALF), jnp.float32),
    mesh=plsc.VectorSubcoreMesh(core_axis_name="c", subcore_axis_name="s"),
    scratch_types=[
        pltpu.VMEM((BLOCK_E,), jnp.int32),
        pltpu.VMEM((BLOCK_E,), jnp.int32),
        pltpu.VMEM((BLOCK_E, D_HALF), jnp.float32),
        pltpu.VMEM_SHARED((N_NODES, D_HALF), jnp.float32),
        pltpu.SemaphoreType.DMA,
    ],
)(_sc_body)


def _scale_body(a_ref, ci_ref, o_ref):
    o_ref[:, :D_HALF] = a_ref[0] * ci_ref[...]
    o_ref[:, D_HALF:] = a_ref[1] * ci_ref[...]


_tc_scale = pl.pallas_call(
    _scale_body,
    grid=(N_NODES // _R,),
    in_specs=[
        pl.BlockSpec((N_CORES, _R, D_HALF), lambda i: (0, i, 0)),
        pl.BlockSpec((_R, 1), lambda i: (i, 0)),
    ],
    out_specs=pl.BlockSpec((_R, D_FEAT), lambda i: (i, 0)),
    out_shape=jax.ShapeDtypeStruct((N_NODES, D_FEAT), jnp.float32),
)


def kernel(src_feats, edge_index, cj, ci):
    src = edge_index[0].astype(jnp.int32)
    dst = edge_index[1].astype(jnp.int32)
    weighted = _tc_weight(src_feats, cj).reshape(N_CORES * N_NODES, D_HALF)
    zeros = jnp.zeros((N_NODES, D_HALF), jnp.float32)
    agg = _sc_gather_scatter(weighted, src, dst, zeros)
    return _tc_scale(agg.reshape(N_CORES, N_NODES, D_HALF), ci)


# speedup vs baseline: 3.6410x; 3.6410x over previous
import functools

import jax
import jax.numpy as jnp
from jax import lax
from jax.experimental import pallas as pl
from jax.experimental.pallas import tpu as pltpu
from jax.experimental.pallas import tpu_sc as plsc

N_NODES = 10000
N_EDGES = 160000
D_FEAT = 256
D_HALF = 128
N_CORES = 2
N_TILES = 16

EDGES_PER_TILE = N_EDGES // N_TILES
BLOCK_E = 80
N_BLOCKS = EDGES_PER_TILE // BLOCK_E
N_PAD = 10240
ROWS_PER_TILE = N_PAD // N_TILES


def _weight_body(x_ref, cj_ref, w_ref):
    w_ref[0] = x_ref[:, :D_HALF] * cj_ref[...]
    w_ref[1] = x_ref[:, D_HALF:] * cj_ref[...]


_R = 1000

_tc_weight = pl.pallas_call(
    _weight_body,
    grid=(N_NODES // _R,),
    in_specs=[
        pl.BlockSpec((_R, D_FEAT), lambda i: (i, 0)),
        pl.BlockSpec((_R, 1), lambda i: (i, 0)),
    ],
    out_specs=pl.BlockSpec((N_CORES, _R, D_HALF), lambda i: (0, i, 0)),
    out_shape=jax.ShapeDtypeStruct((N_CORES, N_NODES, D_HALF), jnp.float32),
)


def _sc_body(w_hbm, src_hbm, dst_hbm, zero_hbm, out_hbm,
             src_v, dst_v, rows_v, acc, sem):
    c = lax.axis_index("c")
    s = lax.axis_index("s")
    row_off = c * N_NODES
    out_off = c * N_PAD
    edge_base = s * EDGES_PER_TILE

    r0 = s * ROWS_PER_TILE
    pltpu.sync_copy(zero_hbm.at[pl.ds(r0, ROWS_PER_TILE)],
                    acc.at[pl.ds(r0, ROWS_PER_TILE)])
    plsc.subcore_barrier()

    def block(i, carry):
        off = edge_base + i * BLOCK_E
        pltpu.sync_copy(src_hbm.at[pl.ds(off, BLOCK_E)], src_v)
        pltpu.sync_copy(dst_hbm.at[pl.ds(off, BLOCK_E)], dst_v)
        for j in range(BLOCK_E // 16):
            sl = pl.ds(j * 16, 16)
            src_v[sl] = src_v[sl] + row_off
        pltpu.async_copy(w_hbm.at[src_v], rows_v, sem).wait()
        pltpu.sync_copy(rows_v, acc.at[dst_v], add=True)
        return carry

    lax.fori_loop(0, N_BLOCKS, block, 0)

    plsc.subcore_barrier()
    pltpu.sync_copy(acc.at[pl.ds(r0, ROWS_PER_TILE)],
                    out_hbm.at[pl.ds(out_off + r0, ROWS_PER_TILE)])


_sc_gather_scatter = functools.partial(
    pl.kernel,
    out_type=jax.ShapeDtypeStruct((N_CORES * N_PAD, D_HALF), jnp.float32),
    mesh=plsc.VectorSubcoreMesh(core_axis_name="c", subcore_axis_name="s"),
    scratch_types=[
        pltpu.VMEM((BLOCK_E,), jnp.int32),
        pltpu.VMEM((BLOCK_E,), jnp.int32),
        pltpu.VMEM((BLOCK_E, D_HALF), jnp.float32),
        pltpu.VMEM_SHARED((N_PAD, D_HALF), jnp.float32),
        pltpu.SemaphoreType.DMA,
    ],
)(_sc_body)


def _scale_body(a_ref, ci_ref, o_ref):
    o_ref[:, :D_HALF] = a_ref[0] * ci_ref[...]
    o_ref[:, D_HALF:] = a_ref[1] * ci_ref[...]


_tc_scale = pl.pallas_call(
    _scale_body,
    grid=(N_NODES // _R,),
    in_specs=[
        pl.BlockSpec((N_CORES, _R, D_HALF), lambda i: (0, i, 0)),
        pl.BlockSpec((_R, 1), lambda i: (i, 0)),
    ],
    out_specs=pl.BlockSpec((_R, D_FEAT), lambda i: (i, 0)),
    out_shape=jax.ShapeDtypeStruct((N_NODES, D_FEAT), jnp.float32),
)


def kernel(src_feats, edge_index, cj, ci):
    src = edge_index[0].astype(jnp.int32)
    dst = edge_index[1].astype(jnp.int32)
    weighted = _tc_weight(src_feats, cj).reshape(N_CORES * N_NODES, D_HALF)
    zeros = jnp.zeros((N_PAD, D_HALF), jnp.float32)
    agg = _sc_gather_scatter(weighted, src, dst, zeros)
    return _tc_scale(agg.reshape(N_CORES, N_PAD, D_HALF), ci)


# speedup vs baseline: 4.7886x; 1.3152x over previous
import functools

import jax
import jax.numpy as jnp
from jax import lax
from jax.experimental import pallas as pl
from jax.experimental.pallas import tpu as pltpu
from jax.experimental.pallas import tpu_sc as plsc

N_NODES = 10000
N_EDGES = 160000
D_FEAT = 256
D_HALF = 128
N_CORES = 2
N_TILES = 16

BLOCK_E = 80
N_BLOCKS = 126
EDGES_PER_TILE = N_BLOCKS * BLOCK_E
E_PAD = EDGES_PER_TILE * N_TILES
N_PAD = 10240
ROWS_PER_TILE = N_PAD // N_TILES


def _weight_body(x_ref, cj_ref, w0_ref, w1_ref):
    w0_ref[...] = x_ref[:, :D_HALF] * cj_ref[...]
    w1_ref[...] = x_ref[:, D_HALF:] * cj_ref[...]


_R = 1000

_tc_weight = pl.pallas_call(
    _weight_body,
    grid=(N_NODES // _R,),
    in_specs=[
        pl.BlockSpec((_R, D_FEAT), lambda i: (i, 0)),
        pl.BlockSpec((_R, 1), lambda i: (i, 0)),
    ],
    out_specs=[
        pl.BlockSpec((_R, D_HALF), lambda i: (i, 0)),
        pl.BlockSpec((_R, D_HALF), lambda i: (i, 0)),
    ],
    out_shape=[
        jax.ShapeDtypeStruct((N_NODES, D_HALF), jnp.float32),
        jax.ShapeDtypeStruct((N_NODES, D_HALF), jnp.float32),
    ],
)


def _sc_body(w0_hbm, w1_hbm, idx_hbm, zero_hbm, out_hbm,
             ib0, ib1, rows0, rows1, acc, isem0, isem1, gsem0, gsem1):
    c = lax.axis_index("c")
    s = lax.axis_index("s")
    out_off = c * N_PAD

    def idx_start(i, ib, sem):
        pltpu.async_copy(idx_hbm.at[s, i], ib, sem)

    def idx_wait(ib, sem):
        pltpu.make_async_copy(idx_hbm.at[0, 0], ib, sem).wait()

    def start_gather(ib, rows, sem):
        @pl.when(c == 0)
        def _():
            pltpu.async_copy(w0_hbm.at[ib.at[0]], rows, sem)

        @pl.when(c == 1)
        def _():
            pltpu.async_copy(w1_hbm.at[ib.at[0]], rows, sem)

    def wait_gather(rows, sem):
        pltpu.make_async_copy(w0_hbm.at[pl.ds(0, BLOCK_E)], rows, sem).wait()

    def scatter(ib, rows):
        pltpu.sync_copy(rows, acc.at[ib.at[1]], add=True)

    idx_start(0, ib0, isem0)
    idx_start(1, ib1, isem1)
    r0 = s * ROWS_PER_TILE
    pltpu.sync_copy(zero_hbm.at[pl.ds(r0, ROWS_PER_TILE)],
                    acc.at[pl.ds(r0, ROWS_PER_TILE)])
    plsc.subcore_barrier()

    idx_wait(ib0, isem0)
    start_gather(ib0, rows0, gsem0)

    def block2(k, carry):
        i = 2 * k
        idx_wait(ib1, isem1)
        wait_gather(rows0, gsem0)
        start_gather(ib1, rows1, gsem1)
        scatter(ib0, rows0)

        @pl.when(i + 2 < N_BLOCKS)
        def _():
            idx_start(i + 2, ib0, isem0)
            idx_wait(ib0, isem0)

        wait_gather(rows1, gsem1)

        @pl.when(i + 2 < N_BLOCKS)
        def _():
            start_gather(ib0, rows0, gsem0)

        scatter(ib1, rows1)

        @pl.when(i + 3 < N_BLOCKS)
        def _():
            idx_start(i + 3, ib1, isem1)

        return carry

    lax.fori_loop(0, N_BLOCKS // 2, block2, 0)

    plsc.subcore_barrier()
    pltpu.sync_copy(acc.at[pl.ds(r0, ROWS_PER_TILE)],
                    out_hbm.at[pl.ds(out_off + r0, ROWS_PER_TILE)])


_sc_gather_scatter = functools.partial(
    pl.kernel,
    out_type=jax.ShapeDtypeStruct((N_CORES * N_PAD, D_HALF), jnp.float32),
    mesh=plsc.VectorSubcoreMesh(core_axis_name="c", subcore_axis_name="s"),
    scratch_types=[
        pltpu.VMEM((2, BLOCK_E), jnp.int32),
        pltpu.VMEM((2, BLOCK_E), jnp.int32),
        pltpu.VMEM((BLOCK_E, D_HALF), jnp.float32),
        pltpu.VMEM((BLOCK_E, D_HALF), jnp.float32),
        pltpu.VMEM_SHARED((N_PAD, D_HALF), jnp.float32),
        pltpu.SemaphoreType.DMA,
        pltpu.SemaphoreType.DMA,
        pltpu.SemaphoreType.DMA,
        pltpu.SemaphoreType.DMA,
    ],
)(_sc_body)


def _scale_body(a_ref, ci_ref, o_ref):
    o_ref[:, :D_HALF] = a_ref[0] * ci_ref[...]
    o_ref[:, D_HALF:] = a_ref[1] * ci_ref[...]


_tc_scale = pl.pallas_call(
    _scale_body,
    grid=(N_NODES // _R,),
    in_specs=[
        pl.BlockSpec((N_CORES, _R, D_HALF), lambda i: (0, i, 0)),
        pl.BlockSpec((_R, 1), lambda i: (i, 0)),
    ],
    out_specs=pl.BlockSpec((_R, D_FEAT), lambda i: (i, 0)),
    out_shape=jax.ShapeDtypeStruct((N_NODES, D_FEAT), jnp.float32),
)


def kernel(src_feats, edge_index, cj, ci):
    src = edge_index[0].astype(jnp.int32)
    dst = edge_index[1].astype(jnp.int32)
    pad = E_PAD - N_EDGES
    src3 = jnp.concatenate([src, jnp.zeros((pad,), jnp.int32)])
    src3 = src3.reshape(N_TILES, N_BLOCKS, BLOCK_E)
    dst3 = jnp.concatenate([dst, jnp.full((pad,), N_NODES, jnp.int32)])
    dst3 = dst3.reshape(N_TILES, N_BLOCKS, BLOCK_E)
    idx = jnp.stack([src3, dst3], axis=2)

    w0, w1 = _tc_weight(src_feats, cj)
    zeros = jnp.zeros((N_PAD, D_HALF), jnp.float32)
    agg = _sc_gather_scatter(w0, w1, idx, zeros)
    return _tc_scale(agg.reshape(N_CORES, N_PAD, D_HALF), ci)
